# baseline (speedup 1.0000x reference)
import functools

import jax
import jax.numpy as jnp
from jax import lax
from jax.experimental import pallas as pl
from jax.experimental.pallas import tpu as pltpu
from jax.experimental.pallas import tpu_sc as plsc

B = 8192
D = 28
H = 128
L = 256
K = 8192

NSPLIT = 2
BM = 256
BN = 512
BM2 = 512

_NC = 2
_NS = 16
_NW = _NC * _NS


def _argmin_body(z2_ref, z_ref, e2_ref, cbt_ref, out_ref):
    z = z_ref[...]
    z2 = z2_ref[...]
    zz = z + z
    run_val = jnp.full((BM, 128), jnp.inf, dtype=jnp.float32)
    run_idx = jnp.zeros((BM, 128), dtype=jnp.float32)
    io128 = lax.broadcasted_iota(jnp.int32, (1, 128), 1).astype(jnp.float32)
    for c in range(K // BN):
        cbt = cbt_ref[:, pl.ds(c * BN, BN)]
        ze2 = lax.dot_general(zz, cbt, (((1,), (0,)), ((), ())),
                              preferred_element_type=jnp.float32)
        for g in range(BN // 128):
            j0 = c * BN + g * 128
            e2g = e2_ref[:, j0:j0 + 128]
            dg = (z2 + e2g) - ze2[:, g * 128:(g + 1) * 128]
            upd = dg < run_val
            run_val = jnp.minimum(dg, run_val)
            run_idx = jnp.where(upd, io128 + float(j0), run_idx)
    m = jnp.min(run_val, axis=1, keepdims=True)
    loc = jnp.min(jnp.where(run_val == m, run_idx, 3e9), axis=1, keepdims=True)
    out_ref[...] = loc.astype(jnp.int32)


def _nearest_indices(z, z2, e2, cbt):
    rows = z.shape[0]
    return pl.pallas_call(
        _argmin_body,
        grid=(rows // BM,),
        in_specs=[
            pl.BlockSpec((BM, 1), lambda i: (i, 0)),
            pl.BlockSpec((BM, L), lambda i: (i, 0)),
            pl.BlockSpec((1, K), lambda i: (0, 0)),
            pl.BlockSpec((L, K), lambda i: (0, 0)),
        ],
        out_specs=pl.BlockSpec((BM, 1), lambda i: (i, 0)),
        out_shape=jax.ShapeDtypeStruct((rows, 1), jnp.int32),
    )(z2, z, e2, cbt)


@functools.cache
def _make_sc_gather(rows):
    bpw = rows // _NW

    @functools.partial(
        pl.kernel,
        out_type=jax.ShapeDtypeStruct((rows, L), jnp.float32),
        mesh=plsc.VectorSubcoreMesh(core_axis_name="c", subcore_axis_name="s"),
        scratch_types=[
            pltpu.VMEM((bpw,), jnp.int32),
            pltpu.VMEM((bpw, L), jnp.float32),
            pltpu.SemaphoreType.DMA,
        ],
    )
    def _sc_gather_kernel(table_hbm, idx_hbm, out_hbm, idx_v, rows_v, sem):
        wid = lax.axis_index("s") * _NC + lax.axis_index("c")
        base = wid * bpw
        pltpu.sync_copy(idx_hbm.at[pl.ds(base, bpw)], idx_v)
        pltpu.async_copy(table_hbm.at[idx_v], rows_v, sem).wait()
        pltpu.sync_copy(rows_v, out_hbm.at[pl.ds(base, bpw)])

    return _sc_gather_kernel


def _sc_gather(table, idx):
    return _make_sc_gather(idx.shape[0])(table, idx)


def _decoder_body(x_ref, z_ref, q_ref, w1_ref, b1_ref, w2_ref, b2_ref,
                  recon_ref, sq_ref, sr_ref):
    i = pl.program_id(0)
    q = q_ref[...]
    z = z_ref[...]
    x = x_ref[...]
    hd = jnp.maximum(
        lax.dot_general(q, w1_ref[...], (((1,), (0,)), ((), ())),
                        preferred_element_type=jnp.float32) + b1_ref[...],
        0.0)
    recon = lax.dot_general(hd, w2_ref[...], (((1,), (0,)), ((), ())),
                            preferred_element_type=jnp.float32) + b2_ref[...]
    recon_ref[...] = recon
    psq = jnp.sum((z - q) ** 2).reshape(1, 1)
    psr = jnp.sum((recon - x) ** 2).reshape(1, 1)

    @pl.when(i == 0)
    def _():
        sq_ref[...] = psq
        sr_ref[...] = psr

    @pl.when(i > 0)
    def _():
        sq_ref[...] += psq
        sr_ref[...] += psr


def _decode(x, z, q, dec_w1, dec_b1, dec_w2, dec_b2):
    rows = x.shape[0]
    return pl.pallas_call(
        _decoder_body,
        grid=(rows // BM2,),
        in_specs=[
            pl.BlockSpec((BM2, D), lambda i: (i, 0)),
            pl.BlockSpec((BM2, L), lambda i: (i, 0)),
            pl.BlockSpec((BM2, L), lambda i: (i, 0)),
            pl.BlockSpec((L, H), lambda i: (0, 0)),
            pl.BlockSpec((1, H), lambda i: (0, 0)),
            pl.BlockSpec((H, D), lambda i: (0, 0)),
            pl.BlockSpec((1, D), lambda i: (0, 0)),
        ],
        out_specs=[
            pl.BlockSpec((BM2, D), lambda i: (i, 0)),
            pl.BlockSpec((1, 1), lambda i: (0, 0)),
            pl.BlockSpec((1, 1), lambda i: (0, 0)),
        ],
        out_shape=[
            jax.ShapeDtypeStruct((rows, D), jnp.float32),
            jax.ShapeDtypeStruct((1, 1), jnp.float32),
            jax.ShapeDtypeStruct((1, 1), jnp.float32),
        ],
    )(x, z, q, dec_w1, dec_b1.reshape(1, H), dec_w2, dec_b2.reshape(1, D))


def kernel(x, enc_w1, enc_b1, enc_w2, enc_b2, codebook,
           dec_w1, dec_b1, dec_w2, dec_b2):
    h = jax.nn.relu(x @ enc_w1 + enc_b1)
    z = h @ enc_w2 + enc_b2
    z2 = jnp.sum(z ** 2, axis=1, keepdims=True)
    e2 = jnp.sum(codebook ** 2, axis=1).reshape(1, K)
    cbt = codebook.T

    rows = B // NSPLIT
    recons, sqs, srs = [], [], []
    for s in range(NSPLIT):
        sl = slice(s * rows, (s + 1) * rows)
        idx = _nearest_indices(z[sl], z2[sl], e2, cbt).reshape(rows)
        q = _sc_gather(codebook, idx)
        recon_s, ssq, ssr = _decode(x[sl], z[sl], q,
                                    dec_w1, dec_b1, dec_w2, dec_b2)
        recons.append(recon_s)
        sqs.append(ssq[0, 0])
        srs.append(ssr[0, 0])

    recon = jnp.concatenate(recons, axis=0)
    vq_loss = (sum(sqs) / (B * L)) * 1.25
    recon_loss = sum(srs) / (B * D)
    return (recon, recon_loss, vq_loss)


# speedup vs baseline: 1.1118x; 1.1118x over previous
import functools

import jax
import jax.numpy as jnp
from jax import lax
from jax.experimental import pallas as pl
from jax.experimental.pallas import tpu as pltpu
from jax.experimental.pallas import tpu_sc as plsc

B = 8192
D = 28
H = 128
L = 256
K = 8192

NSPLIT = 1
BM = 256
BN = 512
BM2 = 512

_NC = 2
_NS = 16
_NW = _NC * _NS


def _argmin_body(z2_ref, z_ref, e2_ref, cbt_ref, out_ref):
    zz = z_ref[...]
    z2 = z2_ref[...]
    run_val = jnp.full((BM, 128), jnp.inf, dtype=jnp.float32)
    run_idx = jnp.zeros((BM, 128), dtype=jnp.float32)
    io128 = lax.broadcasted_iota(jnp.int32, (1, 128), 1).astype(jnp.float32)
    for c in range(K // BN):
        cbc = cbt_ref[pl.ds(c * BN, BN), :]
        ze2 = lax.dot_general(zz, cbc, (((1,), (1,)), ((), ())),
                              preferred_element_type=jnp.float32)
        for g in range(BN // 128):
            j0 = c * BN + g * 128
            e2g = e2_ref[:, j0:j0 + 128]
            dg = (z2 + e2g) - ze2[:, g * 128:(g + 1) * 128]
            upd = dg < run_val
            run_val = jnp.minimum(dg, run_val)
            run_idx = jnp.where(upd, io128 + float(j0), run_idx)
    m = jnp.min(run_val, axis=1, keepdims=True)
    loc = jnp.min(jnp.where(run_val == m, run_idx, 3e9), axis=1, keepdims=True)
    out_ref[...] = loc.astype(jnp.int32)


def _nearest_indices(z, z2, e2, cbt):
    rows = z.shape[0]
    return pl.pallas_call(
        _argmin_body,
        grid=(rows // BM,),
        in_specs=[
            pl.BlockSpec((BM, 1), lambda i: (i, 0)),
            pl.BlockSpec((BM, L), lambda i: (i, 0)),
            pl.BlockSpec((1, K), lambda i: (0, 0)),
            pl.BlockSpec((K, L), lambda i: (0, 0)),
        ],
        out_specs=pl.BlockSpec((BM, 1), lambda i: (i, 0)),
        out_shape=jax.ShapeDtypeStruct((rows, 1), jnp.int32),
    )(z2, z, e2, cbt)


@functools.cache
def _make_sc_gather(rows):
    bpw = rows // _NW

    @functools.partial(
        pl.kernel,
        out_type=jax.ShapeDtypeStruct((rows, L), jnp.float32),
        mesh=plsc.VectorSubcoreMesh(core_axis_name="c", subcore_axis_name="s"),
        scratch_types=[
            pltpu.VMEM((bpw,), jnp.int32),
            pltpu.VMEM((bpw, L), jnp.float32),
            pltpu.SemaphoreType.DMA,
        ],
    )
    def _sc_gather_kernel(table_hbm, idx_hbm, out_hbm, idx_v, rows_v, sem):
        wid = lax.axis_index("s") * _NC + lax.axis_index("c")
        base = wid * bpw
        pltpu.sync_copy(idx_hbm.at[pl.ds(base, bpw)], idx_v)
        pltpu.async_copy(table_hbm.at[idx_v], rows_v, sem).wait()
        pltpu.sync_copy(rows_v, out_hbm.at[pl.ds(base, bpw)])

    return _sc_gather_kernel


def _sc_gather(table, idx):
    return _make_sc_gather(idx.shape[0])(table, idx)


def _decoder_body(x_ref, z_ref, q_ref, w1_ref, b1_ref, w2_ref, b2_ref,
                  recon_ref, sq_ref, sr_ref):
    i = pl.program_id(0)
    q = q_ref[...]
    z = z_ref[...]
    x = x_ref[...]
    hd = jnp.maximum(
        lax.dot_general(q, w1_ref[...], (((1,), (0,)), ((), ())),
                        preferred_element_type=jnp.float32) + b1_ref[...],
        0.0)
    recon = lax.dot_general(hd, w2_ref[...], (((1,), (0,)), ((), ())),
                            preferred_element_type=jnp.float32) + b2_ref[...]
    recon_ref[...] = recon
    psq = jnp.sum((z - q) ** 2).reshape(1, 1)
    psr = jnp.sum((recon - x) ** 2).reshape(1, 1)

    @pl.when(i == 0)
    def _():
        sq_ref[...] = psq
        sr_ref[...] = psr

    @pl.when(i > 0)
    def _():
        sq_ref[...] += psq
        sr_ref[...] += psr


def _decode(x, z, q, dec_w1, dec_b1, dec_w2, dec_b2):
    rows = x.shape[0]
    return pl.pallas_call(
        _decoder_body,
        grid=(rows // BM2,),
        in_specs=[
            pl.BlockSpec((BM2, D), lambda i: (i, 0)),
            pl.BlockSpec((BM2, L), lambda i: (i, 0)),
            pl.BlockSpec((BM2, L), lambda i: (i, 0)),
            pl.BlockSpec((L, H), lambda i: (0, 0)),
            pl.BlockSpec((1, H), lambda i: (0, 0)),
            pl.BlockSpec((H, D), lambda i: (0, 0)),
            pl.BlockSpec((1, D), lambda i: (0, 0)),
        ],
        out_specs=[
            pl.BlockSpec((BM2, D), lambda i: (i, 0)),
            pl.BlockSpec((1, 1), lambda i: (0, 0)),
            pl.BlockSpec((1, 1), lambda i: (0, 0)),
        ],
        out_shape=[
            jax.ShapeDtypeStruct((rows, D), jnp.float32),
            jax.ShapeDtypeStruct((1, 1), jnp.float32),
            jax.ShapeDtypeStruct((1, 1), jnp.float32),
        ],
    )(x, z, q, dec_w1, dec_b1.reshape(1, H), dec_w2, dec_b2.reshape(1, D))


def kernel(x, enc_w1, enc_b1, enc_w2, enc_b2, codebook,
           dec_w1, dec_b1, dec_w2, dec_b2):
    h = jax.nn.relu(x @ enc_w1 + enc_b1)
    z = h @ enc_w2 + enc_b2
    z2 = jnp.sum(z ** 2, axis=1, keepdims=True)
    e2 = jnp.sum(codebook ** 2, axis=1).reshape(1, K)
    cbt = codebook.astype(jnp.bfloat16)
    zz_bf = (z + z).astype(jnp.bfloat16)

    rows = B // NSPLIT
    recons, sqs, srs = [], [], []
    for s in range(NSPLIT):
        sl = slice(s * rows, (s + 1) * rows)
        idx = _nearest_indices(zz_bf[sl], z2[sl], e2, cbt).reshape(rows)
        q = _sc_gather(codebook, idx)
        recon_s, ssq, ssr = _decode(x[sl], z[sl], q,
                                    dec_w1, dec_b1, dec_w2, dec_b2)
        recons.append(recon_s)
        sqs.append(ssq[0, 0])
        srs.append(ssr[0, 0])

    recon = recons[0] if len(recons) == 1 else jnp.concatenate(recons, axis=0)
    vq_loss = (sum(sqs) / (B * L)) * 1.25
    recon_loss = sum(srs) / (B * D)
    return (recon, recon_loss, vq_loss)


# speedup vs baseline: 1.1775x; 1.0591x over previous
import functools

import jax
import jax.numpy as jnp
from jax import lax
from jax.experimental import pallas as pl
from jax.experimental.pallas import tpu as pltpu
from jax.experimental.pallas import tpu_sc as plsc

B = 8192
D = 28
H = 128
L = 256
K = 8192

NSPLIT = 1
BM = 256
BN = 512
BM2 = 512

_NC = 2
_NS = 16
_NW = _NC * _NS


def _argmin_body(z_ref, e2_ref, cbt_ref, out_ref):
    z = z_ref[...]
    z2 = jnp.sum(z * z, axis=1, keepdims=True)
    zz = (z + z).astype(jnp.bfloat16)
    run_val = jnp.full((BM, 128), jnp.inf, dtype=jnp.float32)
    run_idx = jnp.zeros((BM, 128), dtype=jnp.float32)
    io128 = lax.broadcasted_iota(jnp.int32, (1, 128), 1).astype(jnp.float32)
    for c in range(K // BN):
        cbc = cbt_ref[pl.ds(c * BN, BN), :]
        ze2 = lax.dot_general(zz, cbc, (((1,), (1,)), ((), ())),
                              preferred_element_type=jnp.float32)
        for g in range(BN // 128):
            j0 = c * BN + g * 128
            e2g = e2_ref[:, j0:j0 + 128]
            dg = (z2 + e2g) - ze2[:, g * 128:(g + 1) * 128]
            upd = dg < run_val
            run_val = jnp.minimum(dg, run_val)
            run_idx = jnp.where(upd, io128 + float(j0), run_idx)
    m = jnp.min(run_val, axis=1, keepdims=True)
    loc = jnp.min(jnp.where(run_val == m, run_idx, 3e9), axis=1, keepdims=True)
    out_ref[...] = loc.astype(jnp.int32)


def _nearest_indices(z, e2, cbt):
    rows = z.shape[0]
    return pl.pallas_call(
        _argmin_body,
        grid=(rows // BM,),
        in_specs=[
            pl.BlockSpec((BM, L), lambda i: (i, 0)),
            pl.BlockSpec((1, K), lambda i: (0, 0)),
            pl.BlockSpec((K, L), lambda i: (0, 0)),
        ],
        out_specs=pl.BlockSpec((BM, 1), lambda i: (i, 0)),
        out_shape=jax.ShapeDtypeStruct((rows, 1), jnp.int32),
    )(z, e2, cbt)


@functools.cache
def _make_sc_gather(rows):
    bpw = rows // _NW

    @functools.partial(
        pl.kernel,
        out_type=jax.ShapeDtypeStruct((rows, L), jnp.float32),
        mesh=plsc.VectorSubcoreMesh(core_axis_name="c", subcore_axis_name="s"),
        scratch_types=[
            pltpu.VMEM((bpw,), jnp.int32),
            pltpu.VMEM((bpw, L), jnp.float32),
            pltpu.SemaphoreType.DMA,
        ],
    )
    def _sc_gather_kernel(table_hbm, idx_hbm, out_hbm, idx_v, rows_v, sem):
        wid = lax.axis_index("s") * _NC + lax.axis_index("c")
        base = wid * bpw
        pltpu.sync_copy(idx_hbm.at[pl.ds(base, bpw)], idx_v)
        pltpu.async_copy(table_hbm.at[idx_v], rows_v, sem).wait()
        pltpu.sync_copy(rows_v, out_hbm.at[pl.ds(base, bpw)])

    return _sc_gather_kernel


def _sc_gather(table, idx):
    return _make_sc_gather(idx.shape[0])(table, idx)


def _decoder_body(x_ref, z_ref, q_ref, w1_ref, b1_ref, w2_ref, b2_ref,
                  recon_ref, sq_ref, sr_ref):
    i = pl.program_id(0)
    q = q_ref[...]
    z = z_ref[...]
    x = x_ref[...]
    hd = jnp.maximum(
        lax.dot_general(q, w1_ref[...], (((1,), (0,)), ((), ())),
                        preferred_element_type=jnp.float32) + b1_ref[...],
        0.0)
    recon = lax.dot_general(hd, w2_ref[...], (((1,), (0,)), ((), ())),
                            preferred_element_type=jnp.float32) + b2_ref[...]
    recon_ref[...] = recon
    psq = jnp.sum((z - q) ** 2).reshape(1, 1)
    psr = jnp.sum((recon - x) ** 2).reshape(1, 1)

    @pl.when(i == 0)
    def _():
        sq_ref[...] = psq
        sr_ref[...] = psr

    @pl.when(i > 0)
    def _():
        sq_ref[...] += psq
        sr_ref[...] += psr


def _decode(x, z, q, dec_w1, dec_b1, dec_w2, dec_b2):
    rows = x.shape[0]
    return pl.pallas_call(
        _decoder_body,
        grid=(rows // BM2,),
        in_specs=[
            pl.BlockSpec((BM2, D), lambda i: (i, 0)),
            pl.BlockSpec((BM2, L), lambda i: (i, 0)),
            pl.BlockSpec((BM2, L), lambda i: (i, 0)),
            pl.BlockSpec((L, H), lambda i: (0, 0)),
            pl.BlockSpec((1, H), lambda i: (0, 0)),
            pl.BlockSpec((H, D), lambda i: (0, 0)),
            pl.BlockSpec((1, D), lambda i: (0, 0)),
        ],
        out_specs=[
            pl.BlockSpec((BM2, D), lambda i: (i, 0)),
            pl.BlockSpec((1, 1), lambda i: (0, 0)),
            pl.BlockSpec((1, 1), lambda i: (0, 0)),
        ],
        out_shape=[
            jax.ShapeDtypeStruct((rows, D), jnp.float32),
            jax.ShapeDtypeStruct((1, 1), jnp.float32),
            jax.ShapeDtypeStruct((1, 1), jnp.float32),
        ],
    )(x, z, q, dec_w1, dec_b1.reshape(1, H), dec_w2, dec_b2.reshape(1, D))


def kernel(x, enc_w1, enc_b1, enc_w2, enc_b2, codebook,
           dec_w1, dec_b1, dec_w2, dec_b2):
    h = jax.nn.relu(x @ enc_w1 + enc_b1)
    z = h @ enc_w2 + enc_b2
    e2 = jnp.sum(codebook ** 2, axis=1).reshape(1, K)
    cbt = codebook.astype(jnp.bfloat16)

    rows = B // NSPLIT
    recons, sqs, srs = [], [], []
    for s in range(NSPLIT):
        sl = slice(s * rows, (s + 1) * rows)
        idx = _nearest_indices(z[sl], e2, cbt).reshape(rows)
        q = _sc_gather(codebook, idx)
        recon_s, ssq, ssr = _decode(x[sl], z[sl], q,
                                    dec_w1, dec_b1, dec_w2, dec_b2)
        recons.append(recon_s)
        sqs.append(ssq[0, 0])
        srs.append(ssr[0, 0])

    recon = recons[0] if len(recons) == 1 else jnp.concatenate(recons, axis=0)
    vq_loss = (sum(sqs) / (B * L)) * 1.25
    recon_loss = sum(srs) / (B * D)
    return (recon, recon_loss, vq_loss)
